# baseline (speedup 1.0000x reference)
import functools
import math

import jax
import jax.numpy as jnp
from jax import lax
from jax.experimental import pallas as pl
from jax.experimental.pallas import tpu as pltpu
from jax.experimental.pallas import tpu_sc as plsc

_K = 10
_ALPHA = 0.1
_T = 16
_B = 128
_L = 16
_NB = 3


def _round_up(v, m):
    return (v + m - 1) // m * m


def kernel(x, edge_index, W1, b1, W2, b2):
    N, F = x.shape
    H = W1.shape[1]
    C = W2.shape[1]
    E = edge_index.shape[1]
    assert C == _L

    R = _round_up(math.ceil((N + 1) / _T), _L)
    Npad = R * _T
    CH = _round_up(math.ceil(E / (_T * _B)), 2 * _NB)
    Epad = _T * CH * _B
    CHs = CH + _NB

    src = edge_index[0]
    dst = edge_index[1]
    pad_e = Epad - E
    srcp = jnp.concatenate([src, jnp.zeros((pad_e,), src.dtype)]).reshape(_T, CH, _B)
    srcp = jnp.pad(srcp, ((0, 0), (0, _NB), (0, 0)))
    dstp = jnp.concatenate([dst, jnp.full((pad_e,), N, dst.dtype)]).reshape(_T, CH, _B)
    x_pad = jnp.pad(x, ((0, Npad - N), (0, 0)))

    mesh = plsc.VectorSubcoreMesh(core_axis_name="c", subcore_axis_name="s",
                                  num_cores=1)
    sc_params = pltpu.CompilerParams(needs_layout_passes=False,
                                     use_tc_tiling_on_sc=False)

    @functools.partial(
        pl.kernel,
        out_type=jax.ShapeDtypeStruct((Npad,), jnp.float32),
        mesh=mesh,
        compiler_params=sc_params,
        scratch_types=[
            pltpu.VMEM((CH * _B,), jnp.int32),
            pltpu.VMEM((Npad,), jnp.float32),
            pltpu.VMEM((_T, R), jnp.float32),
            pltpu.VMEM((R,), jnp.float32),
            pltpu.VMEM_SHARED((_T, Npad), jnp.float32),
        ],
    )
    def deg_kernel(dst_hbm, deg_hbm, dst_v, hist_v, blk_v, out_v, part_sp):
        sid = lax.axis_index("s")
        pltpu.sync_copy(dst_hbm.at[sid], dst_v)

        def zbody(i, c):
            for u in range(8):
                hist_v[pl.ds((i * 8 + u) * _L, _L)] = jnp.zeros((_L,), jnp.float32)
            return c
        lax.fori_loop(0, Npad // (8 * _L), zbody, 0)

        ones = jnp.ones((_L,), jnp.float32)

        def hbody(j, c):
            for u in range(8):
                idx = dst_v[pl.ds((j * 8 + u) * _L, _L)]
                plsc.addupdate_scatter(hist_v, [idx], ones)
            return c
        lax.fori_loop(0, (CH * _B) // (8 * _L), hbody, 0)

        pltpu.sync_copy(hist_v, part_sp.at[sid])
        plsc.subcore_barrier()
        for r in range(_T):
            pltpu.sync_copy(part_sp.at[r, pl.ds(sid * R, R)], blk_v.at[r])

        def cbody(c, carry):
            acc = blk_v[0, pl.ds(c * _L, _L)]
            for r in range(1, _T):
                acc = acc + blk_v[r, pl.ds(c * _L, _L)]
            out_v[pl.ds(c * _L, _L)] = acc + 1.0
            return carry
        lax.fori_loop(0, R // _L, cbody, 0)
        pltpu.sync_copy(out_v, deg_hbm.at[pl.ds(sid * R, R)])

    deg = deg_kernel(dstp.reshape(_T, CH * _B))

    @functools.partial(
        pl.pallas_call,
        grid=(Npad // R,),
        in_specs=[
            pl.BlockSpec((R, F), lambda i: (i, 0)),
            pl.BlockSpec((F, H), lambda i: (0, 0)),
            pl.BlockSpec((1, H), lambda i: (0, 0)),
            pl.BlockSpec((H, C), lambda i: (0, 0)),
            pl.BlockSpec((1, C), lambda i: (0, 0)),
        ],
        out_specs=pl.BlockSpec((R, C), lambda i: (i, 0)),
        out_shape=jax.ShapeDtypeStruct((Npad, C), jnp.float32),
    )
    def mlp_kernel(x_ref, w1_ref, b1_ref, w2_ref, b2_ref, h_ref):
        h1 = jnp.dot(x_ref[...], w1_ref[...],
                     preferred_element_type=jnp.float32) + b1_ref[...]
        h1 = jnp.maximum(h1, 0.0)
        h_ref[...] = jnp.dot(h1, w2_ref[...],
                             preferred_element_type=jnp.float32) + b2_ref[...]

    h = mlp_kernel(x_pad, W1, b1.reshape(1, H), W2, b2.reshape(1, C))

    @functools.partial(
        pl.kernel,
        out_type=jax.ShapeDtypeStruct((Npad, C), jnp.float32),
        mesh=mesh,
        compiler_params=sc_params,
        scratch_types=[
            pltpu.VMEM((CHs, _B), jnp.int32),
            pltpu.VMEM((CH, _B), jnp.int32),
            pltpu.VMEM((R, C), jnp.float32),
            pltpu.VMEM((R, C), jnp.float32),
            pltpu.VMEM((R, C), jnp.float32),
            pltpu.VMEM((R, C), jnp.float32),
            pltpu.VMEM((2 * _NB * _B, C), jnp.float32),
            pltpu.VMEM((R,), jnp.float32),
            pltpu.VMEM((R,), jnp.float32),
            pltpu.VMEM((R,), jnp.float32),
            pltpu.VMEM_SHARED((Npad, C), jnp.float32),
            pltpu.VMEM_SHARED((Npad, C), jnp.float32),
            pltpu.SemaphoreType.DMA,
        ],
    )
    def prop_kernel(src_hbm, dst_hbm, h_hbm, deg_hbm, zt_hbm,
                    src_v, dst_v, zt_own, zt0_own, a_own, agg_own, rows,
                    deg_v, dinv_v, ab1_v, agg_sp, zt_sp, gsem):
        sid = lax.axis_index("s")
        base = sid * R
        pltpu.sync_copy(src_hbm.at[sid], src_v)
        pltpu.sync_copy(dst_hbm.at[sid], dst_v)
        pltpu.sync_copy(h_hbm.at[pl.ds(base, R)], zt0_own)
        pltpu.sync_copy(deg_hbm.at[pl.ds(base, R)], deg_v)

        def dbody(i, c):
            d = deg_v[pl.ds(i * _L, _L)]
            yi = jnp.int32(0x5F3759DF) - lax.shift_right_logical(
                lax.bitcast_convert_type(d, jnp.int32), 1)
            y = lax.bitcast_convert_type(yi, jnp.float32)
            for _ in range(3):
                y = y * (1.5 - 0.5 * d * y * y)
            dinv_v[pl.ds(i * _L, _L)] = y
            ab1_v[pl.ds(i * _L, _L)] = (1.0 - _ALPHA) / d
            return c
        lax.fori_loop(0, R // _L, dbody, 0)

        def ebody(r8, c):
            for u in range(8):
                r = r8 * 8 + u
                idx = jnp.broadcast_to(r, (_L,))
                dv = plsc.load_gather(dinv_v, [idx])
                av = plsc.load_gather(ab1_v, [idx])
                z0 = zt0_own[r] * dv
                zt0_own[r] = z0
                a_own[r] = av
                zt_own[r] = z0
            return c
        lax.fori_loop(0, R // 8, ebody, 0)

        pltpu.sync_copy(zt_own, zt_sp.at[pl.ds(base, R)])
        pltpu.sync_copy(zt_own, agg_sp.at[pl.ds(base, R)])
        plsc.subcore_barrier()

        def buf(b):
            return rows.at[pl.ds(b * _B, _B)]

        def fire(j, b):
            pltpu.async_copy(zt_sp.at[src_v.at[j]], buf(b), gsem)

        def drain(b):
            pltpu.make_async_copy(zt_hbm.at[pl.ds(0, _B)], buf(b), gsem).wait()

        def scat(j, b):
            pltpu.sync_copy(buf(b), agg_sp.at[dst_v.at[j]], add=True)

        def kbody(k, carry):
            for b in range(_NB):
                fire(b, b)

            def gbody(p, c):
                gb = p * 2 * _NB
                for b in range(_NB):
                    fire(gb + _NB + b, _NB + b)
                for b in range(_NB):
                    drain(b)
                    scat(gb + b, b)
                for b in range(_NB):
                    fire(gb + 2 * _NB + b, b)
                for b in range(_NB):
                    drain(_NB + b)
                    scat(gb + _NB + b, _NB + b)
                return c
            lax.fori_loop(0, CH // (2 * _NB), gbody, 0)
            for b in range(_NB):
                drain(b)
            plsc.subcore_barrier()

            pltpu.sync_copy(agg_sp.at[pl.ds(base, R)], agg_own)

            def rbody(r8, c):
                for u in range(8):
                    r = r8 * 8 + u
                    zt_own[r] = a_own[r] * agg_own[r] + _ALPHA * zt0_own[r]
                return c
            lax.fori_loop(0, R // 8, rbody, 0)
            pltpu.sync_copy(zt_own, zt_sp.at[pl.ds(base, R)])
            pltpu.sync_copy(zt_own, agg_sp.at[pl.ds(base, R)])
            plsc.subcore_barrier()
            return carry
        lax.fori_loop(0, _K, kbody, 0)
        pltpu.sync_copy(zt_own, zt_hbm.at[pl.ds(base, R)])

    ztK = prop_kernel(srcp, dstp, h, deg)

    @functools.partial(
        pl.pallas_call,
        out_shape=jax.ShapeDtypeStruct((Npad, C), jnp.float32),
    )
    def out_kernel(zt_ref, deg_ref, o_ref):
        z = zt_ref[...] * jnp.sqrt(deg_ref[...])
        m = jnp.max(z, axis=1, keepdims=True)
        e = jnp.exp(z - m)
        s = jnp.sum(e, axis=1, keepdims=True)
        o_ref[...] = z - (jnp.log(s) + m)

    out = out_kernel(ztK, deg.reshape(Npad, 1))
    return out[:N]


# speedup vs baseline: 1.0860x; 1.0860x over previous
import functools
import math

import jax
import jax.numpy as jnp
from jax import lax
from jax.experimental import pallas as pl
from jax.experimental.pallas import tpu as pltpu
from jax.experimental.pallas import tpu_sc as plsc

_K = 10
_ALPHA = 0.1
_T = 16
_B = 128
_L = 16
_NB = 2


def _round_up(v, m):
    return (v + m - 1) // m * m


def kernel(x, edge_index, W1, b1, W2, b2):
    N, F = x.shape
    H = W1.shape[1]
    C = W2.shape[1]
    E = edge_index.shape[1]
    assert C == _L

    R = _round_up(math.ceil((N + 1) / _T), _L)
    Npad = R * _T
    CH = _round_up(math.ceil(E / (_T * _B)), 2 * _NB)
    Epad = _T * CH * _B
    CHs = CH + _NB

    src = edge_index[0]
    dst = edge_index[1]
    pad_e = Epad - E
    srcp = jnp.concatenate([src, jnp.zeros((pad_e,), src.dtype)]).reshape(_T, CH, _B)
    srcp = jnp.pad(srcp, ((0, 0), (0, _NB), (0, 0)))
    dstp = jnp.concatenate([dst, jnp.full((pad_e,), N, dst.dtype)]).reshape(_T, CH, _B)
    x_pad = jnp.pad(x, ((0, Npad - N), (0, 0)))

    mesh = plsc.VectorSubcoreMesh(core_axis_name="c", subcore_axis_name="s",
                                  num_cores=1)
    sc_params = pltpu.CompilerParams(needs_layout_passes=False,
                                     use_tc_tiling_on_sc=False)

    @functools.partial(
        pl.kernel,
        out_type=jax.ShapeDtypeStruct((Npad,), jnp.float32),
        mesh=mesh,
        compiler_params=sc_params,
        scratch_types=[
            pltpu.VMEM((CH * _B,), jnp.int32),
            pltpu.VMEM((Npad,), jnp.float32),
            pltpu.VMEM((_T, R), jnp.float32),
            pltpu.VMEM((R,), jnp.float32),
            pltpu.VMEM_SHARED((_T, Npad), jnp.float32),
        ],
    )
    def deg_kernel(dst_hbm, deg_hbm, dst_v, hist_v, blk_v, out_v, part_sp):
        sid = lax.axis_index("s")
        pltpu.sync_copy(dst_hbm.at[sid], dst_v)

        def zbody(i, c):
            for u in range(8):
                hist_v[pl.ds((i * 8 + u) * _L, _L)] = jnp.zeros((_L,), jnp.float32)
            return c
        lax.fori_loop(0, Npad // (8 * _L), zbody, 0)

        ones = jnp.ones((_L,), jnp.float32)

        def hbody(j, c):
            for u in range(8):
                idx = dst_v[pl.ds((j * 8 + u) * _L, _L)]
                plsc.addupdate_scatter(hist_v, [idx], ones)
            return c
        lax.fori_loop(0, (CH * _B) // (8 * _L), hbody, 0)

        pltpu.sync_copy(hist_v, part_sp.at[sid])
        plsc.subcore_barrier()
        for r in range(_T):
            pltpu.sync_copy(part_sp.at[r, pl.ds(sid * R, R)], blk_v.at[r])

        def cbody(c, carry):
            acc = blk_v[0, pl.ds(c * _L, _L)]
            for r in range(1, _T):
                acc = acc + blk_v[r, pl.ds(c * _L, _L)]
            out_v[pl.ds(c * _L, _L)] = acc + 1.0
            return carry
        lax.fori_loop(0, R // _L, cbody, 0)
        pltpu.sync_copy(out_v, deg_hbm.at[pl.ds(sid * R, R)])

    deg = deg_kernel(dstp.reshape(_T, CH * _B))

    @functools.partial(
        pl.pallas_call,
        grid=(Npad // R,),
        in_specs=[
            pl.BlockSpec((R, F), lambda i: (i, 0)),
            pl.BlockSpec((F, H), lambda i: (0, 0)),
            pl.BlockSpec((1, H), lambda i: (0, 0)),
            pl.BlockSpec((H, C), lambda i: (0, 0)),
            pl.BlockSpec((1, C), lambda i: (0, 0)),
        ],
        out_specs=pl.BlockSpec((R, C), lambda i: (i, 0)),
        out_shape=jax.ShapeDtypeStruct((Npad, C), jnp.float32),
    )
    def mlp_kernel(x_ref, w1_ref, b1_ref, w2_ref, b2_ref, h_ref):
        h1 = jnp.dot(x_ref[...], w1_ref[...],
                     preferred_element_type=jnp.float32) + b1_ref[...]
        h1 = jnp.maximum(h1, 0.0)
        h_ref[...] = jnp.dot(h1, w2_ref[...],
                             preferred_element_type=jnp.float32) + b2_ref[...]

    h = mlp_kernel(x_pad, W1, b1.reshape(1, H), W2, b2.reshape(1, C))

    @functools.partial(
        pl.kernel,
        out_type=jax.ShapeDtypeStruct((Npad, C), jnp.float32),
        mesh=mesh,
        compiler_params=sc_params,
        scratch_types=[
            pltpu.VMEM((CHs, _B), jnp.int32),
            pltpu.VMEM((CH, _B), jnp.int32),
            pltpu.VMEM((R, C), jnp.float32),
            pltpu.VMEM((R, C), jnp.float32),
            pltpu.VMEM((R, C), jnp.float32),
            pltpu.VMEM((R, C), jnp.float32),
            pltpu.VMEM((2 * _NB * _B, C), jnp.float32),
            pltpu.VMEM((R,), jnp.float32),
            pltpu.VMEM((R,), jnp.float32),
            pltpu.VMEM((R,), jnp.float32),
            pltpu.VMEM_SHARED((Npad, C), jnp.float32),
            pltpu.VMEM_SHARED((Npad, C), jnp.float32),
            pltpu.SemaphoreType.DMA,
        ],
    )
    def prop_kernel(src_hbm, dst_hbm, h_hbm, deg_hbm, zt_hbm,
                    src_v, dst_v, zt_own, zt0_own, a_own, agg_own, rows,
                    deg_v, dinv_v, ab1_v, agg_sp, zt_sp, gsem):
        sid = lax.axis_index("s")
        base = sid * R
        pltpu.sync_copy(src_hbm.at[sid], src_v)
        pltpu.sync_copy(dst_hbm.at[sid], dst_v)
        pltpu.sync_copy(h_hbm.at[pl.ds(base, R)], zt0_own)
        pltpu.sync_copy(deg_hbm.at[pl.ds(base, R)], deg_v)

        def dbody(i, c):
            d = deg_v[pl.ds(i * _L, _L)]
            yi = jnp.int32(0x5F3759DF) - lax.shift_right_logical(
                lax.bitcast_convert_type(d, jnp.int32), 1)
            y = lax.bitcast_convert_type(yi, jnp.float32)
            for _ in range(3):
                y = y * (1.5 - 0.5 * d * y * y)
            dinv_v[pl.ds(i * _L, _L)] = y
            ab1_v[pl.ds(i * _L, _L)] = (1.0 - _ALPHA) / d
            return c
        lax.fori_loop(0, R // _L, dbody, 0)

        def ebody(r8, c):
            for u in range(8):
                r = r8 * 8 + u
                idx = jnp.broadcast_to(r, (_L,))
                dv = plsc.load_gather(dinv_v, [idx])
                av = plsc.load_gather(ab1_v, [idx])
                z0 = zt0_own[r] * dv
                zt0_own[r] = z0
                a_own[r] = av
                zt_own[r] = z0
            return c
        lax.fori_loop(0, R // 8, ebody, 0)

        pltpu.sync_copy(zt_own, zt_sp.at[pl.ds(base, R)])
        pltpu.sync_copy(zt_own, agg_sp.at[pl.ds(base, R)])
        plsc.subcore_barrier()

        def buf(b):
            return rows.at[pl.ds(b * _B, _B)]

        def fire(j, b):
            pltpu.async_copy(zt_sp.at[src_v.at[j]], buf(b), gsem)

        def drain(b):
            pltpu.make_async_copy(zt_hbm.at[pl.ds(0, _B)], buf(b), gsem).wait()

        def scat(j, b):
            pltpu.sync_copy(buf(b), agg_sp.at[dst_v.at[j]], add=True)

        def kbody(k, carry):
            for b in range(_NB):
                fire(b, b)

            def gbody(p, c):
                gb = p * 2 * _NB
                for b in range(_NB):
                    fire(gb + _NB + b, _NB + b)
                for b in range(_NB):
                    drain(b)
                    scat(gb + b, b)
                for b in range(_NB):
                    fire(gb + 2 * _NB + b, b)
                for b in range(_NB):
                    drain(_NB + b)
                    scat(gb + _NB + b, _NB + b)
                return c
            lax.fori_loop(0, CH // (2 * _NB), gbody, 0)
            for b in range(_NB):
                drain(b)
            plsc.subcore_barrier()

            pltpu.sync_copy(agg_sp.at[pl.ds(base, R)], agg_own)

            def rbody(r8, c):
                for u in range(8):
                    r = r8 * 8 + u
                    zt_own[r] = a_own[r] * agg_own[r] + _ALPHA * zt0_own[r]
                return c
            lax.fori_loop(0, R // 8, rbody, 0)
            pltpu.sync_copy(zt_own, zt_sp.at[pl.ds(base, R)])
            pltpu.sync_copy(zt_own, agg_sp.at[pl.ds(base, R)])
            plsc.subcore_barrier()
            return carry
        lax.fori_loop(0, _K, kbody, 0)
        pltpu.sync_copy(zt_own, zt_hbm.at[pl.ds(base, R)])

    ztK = prop_kernel(srcp, dstp, h, deg)

    @functools.partial(
        pl.pallas_call,
        out_shape=jax.ShapeDtypeStruct((Npad, C), jnp.float32),
    )
    def out_kernel(zt_ref, deg_ref, o_ref):
        z = zt_ref[...] * jnp.sqrt(deg_ref[...])
        m = jnp.max(z, axis=1, keepdims=True)
        e = jnp.exp(z - m)
        s = jnp.sum(e, axis=1, keepdims=True)
        o_ref[...] = z - (jnp.log(s) + m)

    out = out_kernel(ztK, deg.reshape(Npad, 1))
    return out[:N]
